# baseline (speedup 1.0000x reference)
import functools

import jax
import jax.numpy as jnp
from jax import lax
from jax.experimental import pallas as pl
from jax.experimental.pallas import tpu as pltpu
from jax.experimental.pallas import tpu_sc as plsc

_B, _S, _H = 4, 2048, 768
_V, _P, _T = 100000, 2048, 2
_EPS = 1e-12
_NC, _NS = 2, 16
_NW = _NC * _NS
_NTOK = _B * _S
_TPW = _NTOK // _NW
_C = 32
_NCHUNK = _TPW // _C
_CB = _H // 128
_RPC = _C * _CB
_HG = _H // 16
_NG = _C // 16
_PITCH = 17
_NBUF = 3


def _emb_body(idx3, w3, p3, tgb, out,
              ic_all, tgb_v, wv0, wv1, wv2, s1, s2,
              sp0, sp1, sp2, sg0, sg1, sg2, so0, so1, so2):
    wid = lax.axis_index("s") * _NC + lax.axis_index("c")
    pos0 = (wid % (_S // _TPW)) * _TPW
    wv = [wv0, wv1, wv2]
    sp = [sp0, sp1, sp2]
    sg = [sg0, sg1, sg2]
    so = [so0, so1, so2]

    _WS = _TPW * _CB + _TPW
    st0 = pltpu.async_copy(idx3.at[pl.ds(wid * _WS, _WS)],
                           ic_all.at[pl.ds(0, _WS)], sg0)
    st1 = pltpu.async_copy(tgb, tgb_v, so0)
    riota = jnp.arange(16, dtype=jnp.int32)
    riotap = riota * _PITCH
    z = jnp.zeros((16,), jnp.float32)

    def prefill(c):
        b = c % _NBUF
        pb = pos0 + c * _C
        return pltpu.async_copy(p3.at[pl.ds(pb * _CB, _RPC)], wv[b], sp[b])

    def gathers(c):
        b = c % _NBUF
        i0 = c * _RPC
        return [
            pltpu.async_copy(w3.at[ic_all.at[pl.ds(i0, 128)]],
                             wv[b].at[pl.ds(0, 128)], sg[b], add=True),
            pltpu.async_copy(w3.at[ic_all.at[pl.ds(i0 + 128, 64)]],
                             wv[b].at[pl.ds(128, 64)], sg[b], add=True),
        ]

    def writeback(c):
        b = c % _NBUF
        return pltpu.async_copy(wv[b], out.at[pl.ds((wid * _NCHUNK + c) * _RPC,
                                                    _RPC)], so[b])

    def compute(c):
        b = c % _NBUF
        w_v = wv[b]

        def group(gi, _):
            rb0 = 2 * gi * (_CB * 8)
            tf = ic_all[pl.ds(_TPW * _CB + c * _C + gi * 16,
                             16)].astype(jnp.float32)
            tfk = [tf[k] for k in range(16)]

            @plsc.parallel_loop(0, _HG, carry=tuple([z] * 32))
            def p1(hg, cr):
                off = hg % 8
                row0 = rb0 + (hg - off)
                sl = pl.ds(off * 16, 16)
                hsl = pl.ds(hg * 16, 16)
                t0c = tgb_v[0, hsl]
                tdc = tgb_v[1, hsl] - t0c
                acc = []
                for k in range(16):
                    row = row0 + (k // 8) * (_CB * 8) + (k % 8)
                    x = w_v[row, sl] + (t0c + tfk[k] * tdc)
                    w_v[row, sl] = x
                    acc.append((cr[k] + x, cr[16 + k] + x * x))
                return tuple(a for a, _q in acc) + tuple(q for _a, q in acc)

            for k in range(16):
                s1[pl.ds(k * _PITCH, 16)] = p1[k]
                s2[pl.ds(k * _PITCH, 16)] = p1[16 + k]
            asum = z
            asq = z
            for k in range(16):
                asum = asum + plsc.load_gather(s1, [riotap + k])
                asq = asq + plsc.load_gather(s2, [riotap + k])
            mean = asum * (1.0 / _H)
            var = asq * (1.0 / _H) - mean * mean
            xv = var + _EPS
            seed = plsc.bitcast(xv, jnp.int32)
            seed = 0x5F3759DF - lax.shift_right_logical(seed, 1)
            y = plsc.bitcast(seed, jnp.float32)
            for _n in range(3):
                y = y * (1.5 - 0.5 * xv * y * y)
            m2 = mean * y
            ysk = [y[k] for k in range(16)]
            msk = [m2[k] for k in range(16)]

            @plsc.parallel_loop(0, _HG)
            def p2(hg):
                off = hg % 8
                row0 = rb0 + (hg - off)
                sl = pl.ds(off * 16, 16)
                hsl = pl.ds(hg * 16, 16)
                gc = tgb_v[2, hsl]
                bc = tgb_v[3, hsl]
                for k in range(16):
                    row = row0 + (k // 8) * (_CB * 8) + (k % 8)
                    x = w_v[row, sl]
                    w_v[row, sl] = (x * ysk[k] - msk[k]) * gc + bc

            del p2
            return 0

        lax.fori_loop(0, _NG, group, 0)

    pre = {}
    gat = {}
    outs = {}
    pre[0] = prefill(0)
    st0.wait()
    st1.wait()
    pre[0].wait()
    gat[0] = gathers(0)
    pre[1] = prefill(1)
    for c in range(_NCHUNK):
        for cp in gat[c]:
            cp.wait()
        if c + 1 < _NCHUNK:
            pre[c + 1].wait()
            gat[c + 1] = gathers(c + 1)
        if c + 2 < _NCHUNK:
            if c - 1 >= 0:
                outs[c - 1].wait()
            pre[c + 2] = prefill(c + 2)
        compute(c)
        outs[c] = writeback(c)
    for c in range(_NCHUNK - 3, _NCHUNK):
        if c >= 0:
            outs[c].wait()


_mesh = plsc.VectorSubcoreMesh(core_axis_name="c", subcore_axis_name="s")

_emb_kernel = functools.partial(
    pl.kernel,
    mesh=_mesh,
    compiler_params=pltpu.CompilerParams(
        use_tc_tiling_on_sc=False, needs_layout_passes=False),
    out_type=jax.ShapeDtypeStruct((_NTOK * _CB, 128), jnp.float32),
    scratch_types=[
        pltpu.VMEM((_TPW * _CB + _TPW + 16,), jnp.int32),
        pltpu.VMEM((4, _H), jnp.float32),
        pltpu.VMEM((_RPC, 128), jnp.float32),
        pltpu.VMEM((_RPC, 128), jnp.float32),
        pltpu.VMEM((_RPC, 128), jnp.float32),
        pltpu.VMEM((15 * _PITCH + 16,), jnp.float32),
        pltpu.VMEM((15 * _PITCH + 16,), jnp.float32),
        pltpu.SemaphoreType.DMA,
        pltpu.SemaphoreType.DMA,
        pltpu.SemaphoreType.DMA,
        pltpu.SemaphoreType.DMA,
        pltpu.SemaphoreType.DMA,
        pltpu.SemaphoreType.DMA,
        pltpu.SemaphoreType.DMA,
        pltpu.SemaphoreType.DMA,
        pltpu.SemaphoreType.DMA,
    ],
)(_emb_body)


def kernel(input_ids, token_type_ids, word_embeddings, position_embeddings,
           token_type_embeddings, ln_gamma, ln_beta):
    ids = input_ids.reshape(-1).astype(jnp.int32)
    tti = token_type_ids.reshape(-1).astype(jnp.int32)
    w3 = (word_embeddings.reshape(_V // 8, 8, _CB, 128)
          .transpose(0, 2, 1, 3).reshape(_V * _CB, 128))
    p3 = (position_embeddings.reshape(_P // 8, 8, _CB, 128)
          .transpose(0, 2, 1, 3).reshape(_P * _CB, 128))
    v = ids.reshape(-1, _C // 8, 8)
    b3 = (v // 8) * (_CB * 8) + (v % 8)
    idx3 = (b3[:, :, None, :]
            + (jnp.arange(_CB, dtype=jnp.int32) * 8)[None, None, :, None])
    idx3c = jnp.concatenate(
        [idx3.reshape(_NW, _TPW * _CB), tti.reshape(_NW, _TPW)],
        axis=1).reshape(-1)
    tgb = jnp.concatenate(
        [token_type_embeddings, ln_gamma[None, :], ln_beta[None, :]], axis=0)
    out3 = _emb_kernel(idx3c, w3, p3, tgb)
    return (out3.reshape(_NTOK // 8, _CB, 8, 128)
            .transpose(0, 2, 1, 3).reshape(_B, _S, _H))


# speedup vs baseline: 1.0173x; 1.0173x over previous
import functools

import jax
import jax.numpy as jnp
from jax import lax
from jax.experimental import pallas as pl
from jax.experimental.pallas import tpu as pltpu
from jax.experimental.pallas import tpu_sc as plsc

_B, _S, _H = 4, 2048, 768
_V, _P, _T = 100000, 2048, 2
_EPS = 1e-12
_NC, _NS = 2, 16
_NW = _NC * _NS
_NTOK = _B * _S
_TPW = _NTOK // _NW
_C = 32
_NCHUNK = _TPW // _C
_CB = _H // 128
_RPC = _C * _CB
_HG = _H // 16
_NG = _C // 16
_PITCH = 17
_NBUF = 3


def _emb_body(idx3, tti, w3, p3, ttab, gam, bet, out,
              idx_all, tt_all, wv0, wv1, wv2, tt_tab, g_v, b_v, s1, s2,
              sp0, sp1, sp2, sg0, sg1, sg2, so0, so1, so2):
    wid = lax.axis_index("s") * _NC + lax.axis_index("c")
    pos0 = (wid % (_S // _TPW)) * _TPW
    wv = [wv0, wv1, wv2]
    sp = [sp0, sp1, sp2]
    sg = [sg0, sg1, sg2]
    so = [so0, so1, so2]

    st0 = pltpu.async_copy(idx3.at[pl.ds(wid * (_TPW * _CB), _TPW * _CB)],
                           idx_all, sg0)
    st1 = pltpu.async_copy(ttab, tt_tab, so0)
    st2 = pltpu.async_copy(gam, g_v, so1)
    st3 = pltpu.async_copy(bet, b_v, so2)
    st4 = pltpu.async_copy(tti.at[pl.ds(wid * _TPW, _TPW)],
                           tt_all.at[pl.ds(0, _TPW)], sg1)
    riota = jnp.arange(16, dtype=jnp.int32)
    riotap = riota * _PITCH
    z = jnp.zeros((16,), jnp.float32)

    def prefill(c):
        b = c % _NBUF
        pb = pos0 + c * _C
        return pltpu.async_copy(p3.at[pl.ds(pb * _CB, _RPC)], wv[b], sp[b])

    def gathers(c):
        b = c % _NBUF
        i0 = c * _RPC
        return [
            pltpu.async_copy(w3.at[idx_all.at[pl.ds(i0, 128)]],
                             wv[b].at[pl.ds(0, 128)], sg[b], add=True),
            pltpu.async_copy(w3.at[idx_all.at[pl.ds(i0 + 128, 64)]],
                             wv[b].at[pl.ds(128, 64)], sg[b], add=True),
        ]

    def writeback(c):
        b = c % _NBUF
        return pltpu.async_copy(wv[b], out.at[pl.ds((wid * _NCHUNK + c) * _RPC,
                                                    _RPC)], so[b])

    def compute(c):
        b = c % _NBUF
        w_v = wv[b]

        def group(gi, _):
            rb0 = 2 * gi * (_CB * 8)
            tf = tt_all[pl.ds(c * _C + gi * 16, 16)].astype(jnp.float32)
            tfk = [tf[k] for k in range(16)]

            @plsc.parallel_loop(0, _HG, carry=tuple([z] * 32))
            def p1(hg, cr):
                off = hg % 8
                row0 = rb0 + (hg - off)
                sl = pl.ds(off * 16, 16)
                hsl = pl.ds(hg * 16, 16)
                t0c = tt_tab[0, hsl]
                tdc = tt_tab[1, hsl] - t0c
                acc = []
                for k in range(16):
                    row = row0 + (k // 8) * (_CB * 8) + (k % 8)
                    x = w_v[row, sl] + (t0c + tfk[k] * tdc)
                    w_v[row, sl] = x
                    acc.append((cr[k] + x, cr[16 + k] + x * x))
                return tuple(a for a, _q in acc) + tuple(q for _a, q in acc)

            for k in range(16):
                s1[pl.ds(k * _PITCH, 16)] = p1[k]
                s2[pl.ds(k * _PITCH, 16)] = p1[16 + k]
            asum = z
            asq = z
            for k in range(16):
                asum = asum + plsc.load_gather(s1, [riotap + k])
                asq = asq + plsc.load_gather(s2, [riotap + k])
            mean = asum * (1.0 / _H)
            var = asq * (1.0 / _H) - mean * mean
            xv = var + _EPS
            seed = plsc.bitcast(xv, jnp.int32)
            seed = 0x5F3759DF - lax.shift_right_logical(seed, 1)
            y = plsc.bitcast(seed, jnp.float32)
            for _n in range(3):
                y = y * (1.5 - 0.5 * xv * y * y)
            m2 = mean * y
            ysk = [y[k] for k in range(16)]
            msk = [m2[k] for k in range(16)]

            @plsc.parallel_loop(0, _HG)
            def p2(hg):
                off = hg % 8
                row0 = rb0 + (hg - off)
                sl = pl.ds(off * 16, 16)
                hsl = pl.ds(hg * 16, 16)
                gc = g_v[hsl]
                bc = b_v[hsl]
                for k in range(16):
                    row = row0 + (k // 8) * (_CB * 8) + (k % 8)
                    x = w_v[row, sl]
                    w_v[row, sl] = (x * ysk[k] - msk[k]) * gc + bc

            del p2
            return 0

        lax.fori_loop(0, _NG, group, 0)

    pre = {}
    gat = {}
    outs = {}
    pre[0] = prefill(0)
    st0.wait()
    st1.wait()
    st2.wait()
    st3.wait()
    st4.wait()
    pre[0].wait()
    gat[0] = gathers(0)
    pre[1] = prefill(1)
    for c in range(_NCHUNK):
        for cp in gat[c]:
            cp.wait()
        if c + 1 < _NCHUNK:
            pre[c + 1].wait()
            gat[c + 1] = gathers(c + 1)
        if c + 2 < _NCHUNK:
            if c - 1 >= 0:
                outs[c - 1].wait()
            pre[c + 2] = prefill(c + 2)
        compute(c)
        outs[c] = writeback(c)
    for c in range(_NCHUNK - 3, _NCHUNK):
        if c >= 0:
            outs[c].wait()


_mesh = plsc.VectorSubcoreMesh(core_axis_name="c", subcore_axis_name="s")

_emb_kernel = functools.partial(
    pl.kernel,
    mesh=_mesh,
    compiler_params=pltpu.CompilerParams(
        use_tc_tiling_on_sc=False, needs_layout_passes=False),
    out_type=jax.ShapeDtypeStruct((_NTOK * _CB, 128), jnp.float32),
    scratch_types=[
        pltpu.VMEM((_TPW * _CB,), jnp.int32),
        pltpu.VMEM((_TPW + 16,), jnp.int32),
        pltpu.VMEM((_RPC, 128), jnp.float32),
        pltpu.VMEM((_RPC, 128), jnp.float32),
        pltpu.VMEM((_RPC, 128), jnp.float32),
        pltpu.VMEM((_T, _H), jnp.float32),
        pltpu.VMEM((_H,), jnp.float32),
        pltpu.VMEM((_H,), jnp.float32),
        pltpu.VMEM((15 * _PITCH + 16,), jnp.float32),
        pltpu.VMEM((15 * _PITCH + 16,), jnp.float32),
        pltpu.SemaphoreType.DMA,
        pltpu.SemaphoreType.DMA,
        pltpu.SemaphoreType.DMA,
        pltpu.SemaphoreType.DMA,
        pltpu.SemaphoreType.DMA,
        pltpu.SemaphoreType.DMA,
        pltpu.SemaphoreType.DMA,
        pltpu.SemaphoreType.DMA,
        pltpu.SemaphoreType.DMA,
    ],
)(_emb_body)


def kernel(input_ids, token_type_ids, word_embeddings, position_embeddings,
           token_type_embeddings, ln_gamma, ln_beta):
    ids = input_ids.reshape(-1).astype(jnp.int32)
    tti = token_type_ids.reshape(-1).astype(jnp.int32)
    w3 = (word_embeddings.reshape(_V // 8, 8, _CB, 128)
          .transpose(0, 2, 1, 3).reshape(_V * _CB, 128))
    p3 = (position_embeddings.reshape(_P // 8, 8, _CB, 128)
          .transpose(0, 2, 1, 3).reshape(_P * _CB, 128))
    v = ids.reshape(-1, _C // 8, 8)
    b3 = (v // 8) * (_CB * 8) + (v % 8)
    idx3 = (b3[:, :, None, :]
            + (jnp.arange(_CB, dtype=jnp.int32) * 8)[None, None, :, None])
    idx3 = idx3.reshape(-1)
    out3 = _emb_kernel(idx3, tti, w3, p3, token_type_embeddings,
                       ln_gamma, ln_beta)
    return (out3.reshape(_NTOK // 8, _CB, 8, 128)
            .transpose(0, 2, 1, 3).reshape(_B, _S, _H))
